# Initial kernel scaffold; baseline (speedup 1.0000x reference)
import jax
import jax.numpy as jnp
from jax.experimental import pallas as pl


def kernel(logits, labels):
    raise NotImplementedError("write your pallas kernel here")


# speedup vs baseline: 4.5450x; 4.5450x over previous
import functools

import jax
import jax.numpy as jnp
from jax.experimental import pallas as pl
from jax.experimental.pallas import tpu as pltpu

_HARD = 0.9
_SOFT = 0.02
_K = 5


def _loss_body(labels_ref, logits_ref, out_ref, m_ref, s_ref, lab_ref, t5_ref,
               *, V, Vb, NV):
    j = pl.program_id(1)

    @pl.when(j == 0)
    def _init():
        m_ref[...] = jnp.full_like(m_ref, -jnp.inf)
        s_ref[...] = jnp.zeros_like(s_ref)
        lab_ref[...] = jnp.zeros_like(lab_ref)
        t5_ref[...] = jnp.full_like(t5_ref, -jnp.inf)

    x = logits_ref[...]
    Rb = x.shape[0]
    cols = jax.lax.broadcasted_iota(jnp.int32, x.shape, 1) + j * Vb
    x = jnp.where(cols < V, x, -jnp.inf)

    bmax = jnp.max(x, axis=1, keepdims=True)
    m_old = m_ref[...]
    m_new = jnp.maximum(m_old, bmax)
    e = jnp.exp(x - m_new)
    s_ref[...] = s_ref[...] * jnp.exp(m_old - m_new) + jnp.sum(e, axis=1, keepdims=True)
    m_ref[...] = m_new

    lab = labels_ref[...]
    hit = cols == lab
    lab_ref[...] = lab_ref[...] + jnp.sum(jnp.where(hit, x, 0.0), axis=1,
                                          keepdims=True)

    y = x
    w = Vb
    while w > 128:
        w //= 2
        y = jnp.maximum(y[:, :w], y[:, w:2 * w])
    vals = []
    for _ in range(_K):
        v = jnp.max(y, axis=1, keepdims=True)
        vals.append(v)
        y = jnp.where(y >= v, -jnp.inf, y)
    z = jnp.concatenate(vals + [t5_ref[...]], axis=1)
    vals2 = []
    for _ in range(_K):
        v = jnp.max(z, axis=1, keepdims=True)
        vals2.append(v)
        z = jnp.where(z >= v, -jnp.inf, z)
    t5_new = jnp.concatenate(
        vals2 + [jnp.full((Rb, 8 - _K), -jnp.inf, x.dtype)], axis=1)
    t5_ref[...] = t5_new

    @pl.when(j == NV - 1)
    def _finish():
        lse = m_ref[...] + jnp.log(s_ref[...])
        sum5 = jnp.sum(t5_new[:, :_K], axis=1, keepdims=True)
        out_ref[...] = lse - _HARD * lab_ref[...] - _SOFT * sum5


def kernel(logits, labels):
    B, V = logits.shape
    Rb = 256 if B % 256 == 0 else 8
    Vb = 2048 if V >= 2048 else 128
    NV = (V + Vb - 1) // Vb

    labels2 = labels.reshape(B, 1).astype(jnp.int32)
    body = functools.partial(_loss_body, V=V, Vb=Vb, NV=NV)
    out = pl.pallas_call(
        body,
        grid=(B // Rb, NV),
        in_specs=[
            pl.BlockSpec((Rb, 1), lambda i, j: (i, 0)),
            pl.BlockSpec((Rb, Vb), lambda i, j: (i, j)),
        ],
        out_specs=pl.BlockSpec((Rb, 1), lambda i, j: (i, 0)),
        out_shape=jax.ShapeDtypeStruct((B, 1), logits.dtype),
        scratch_shapes=[
            pltpu.VMEM((Rb, 1), jnp.float32),
            pltpu.VMEM((Rb, 1), jnp.float32),
            pltpu.VMEM((Rb, 1), jnp.float32),
            pltpu.VMEM((Rb, 8), jnp.float32),
        ],
        compiler_params=pltpu.CompilerParams(
            dimension_semantics=("parallel", "arbitrary")),
    )(labels2, logits)
    return out.reshape(B)


# speedup vs baseline: 5.2617x; 1.1577x over previous
import functools

import jax
import jax.numpy as jnp
from jax.experimental import pallas as pl
from jax.experimental.pallas import tpu as pltpu

_HARD = 0.9
_SOFT = 0.02
_K = 5


def _loss_body(labels_ref, logits_ref, out_ref, m_ref, s_ref, lab_ref, t5_ref,
               *, V, Vb, NV):
    j = pl.program_id(1)

    @pl.when(j == 0)
    def _init():
        m_ref[...] = jnp.full_like(m_ref, -jnp.inf)
        s_ref[...] = jnp.zeros_like(s_ref)
        lab_ref[...] = jnp.zeros_like(lab_ref)
        t5_ref[...] = jnp.full_like(t5_ref, -jnp.inf)

    x = logits_ref[...]
    Rb = x.shape[0]
    cols = jax.lax.broadcasted_iota(jnp.int32, x.shape, 1) + j * Vb
    x = jnp.where(cols < V, x, -jnp.inf)

    bmax = jnp.max(x, axis=1, keepdims=True)
    m_old = m_ref[...]
    m_new = jnp.maximum(m_old, bmax)
    e = jnp.exp(x - m_new)
    s_ref[...] = s_ref[...] * jnp.exp(m_old - m_new) + jnp.sum(e, axis=1, keepdims=True)
    m_ref[...] = m_new

    lab = labels_ref[...]
    hit = cols == lab
    lab_ref[...] = lab_ref[...] + jnp.sum(jnp.where(hit, x, 0.0), axis=1,
                                          keepdims=True)

    y = x
    w = Vb
    while w > 128:
        w //= 2
        y = jnp.maximum(y[:, :w], y[:, w:2 * w])
    vals = []
    for _ in range(_K):
        v = jnp.max(y, axis=1, keepdims=True)
        vals.append(v)
        y = jnp.where(y >= v, -jnp.inf, y)
    z = jnp.concatenate(vals + [t5_ref[...]], axis=1)
    vals2 = []
    for _ in range(_K):
        v = jnp.max(z, axis=1, keepdims=True)
        vals2.append(v)
        z = jnp.where(z >= v, -jnp.inf, z)
    t5_new = jnp.concatenate(
        vals2 + [jnp.full((Rb, 8 - _K), -jnp.inf, x.dtype)], axis=1)
    t5_ref[...] = t5_new

    @pl.when(j == NV - 1)
    def _finish():
        lse = m_ref[...] + jnp.log(s_ref[...])
        sum5 = jnp.sum(t5_new[:, :_K], axis=1, keepdims=True)
        out_ref[...] = lse - _HARD * lab_ref[...] - _SOFT * sum5


def kernel(logits, labels):
    B, V = logits.shape
    Rb = 128 if B % 128 == 0 else 8
    Vb = 8192 if V >= 8192 else 128
    NV = (V + Vb - 1) // Vb

    labels2 = labels.reshape(B, 1).astype(jnp.int32)
    body = functools.partial(_loss_body, V=V, Vb=Vb, NV=NV)
    out = pl.pallas_call(
        body,
        grid=(B // Rb, NV),
        in_specs=[
            pl.BlockSpec((Rb, 1), lambda i, j: (i, 0)),
            pl.BlockSpec((Rb, Vb), lambda i, j: (i, j)),
        ],
        out_specs=pl.BlockSpec((Rb, 1), lambda i, j: (i, 0)),
        out_shape=jax.ShapeDtypeStruct((B, 1), logits.dtype),
        scratch_shapes=[
            pltpu.VMEM((Rb, 1), jnp.float32),
            pltpu.VMEM((Rb, 1), jnp.float32),
            pltpu.VMEM((Rb, 1), jnp.float32),
            pltpu.VMEM((Rb, 8), jnp.float32),
        ],
        compiler_params=pltpu.CompilerParams(
            dimension_semantics=("parallel", "arbitrary")),
    )(labels2, logits)
    return out.reshape(B)
